# baseline (speedup 1.0000x reference)
import functools

import jax
import jax.numpy as jnp
from jax import lax
from jax.experimental import pallas as pl
from jax.experimental.pallas import tpu as pltpu
from jax.experimental.pallas import tpu_sc as plsc

N_ENT = 100000
N_REL = 16
BATCH = 1024
K = 15
CH = 128
NPAD = 102400
C = NPAD // CH
NB = 1024
GRID = NPAD // NB
GPB = NB // CH
BIG = 1 << 30
CROWS = 16


def _sim_body(q_ref, e_ref, sim_ref, cht_ref, gms_ref):
    i = pl.program_id(0)
    q = q_ref[...]
    e = e_ref[...]
    s = lax.dot_general(q, e, (((1,), (1,)), ((), ())),
                        preferred_element_type=jnp.float32)
    for c in range(GPB):
        sim_ref[c] = s[:, c * CH:(c + 1) * CH]
    st = lax.dot_general(e, q, (((1,), (1,)), ((), ())),
                         preferred_element_type=jnp.float32)
    gms_ref[pl.ds(i * GPB, GPB), :] = st.reshape(GPB, CH, BATCH).max(axis=1)

    @pl.when(i == GRID - 1)
    def _chunksel():
        g = gms_ref[...]
        iota_c = lax.broadcasted_iota(jnp.int32, (C, BATCH), 0)
        for j in range(CROWS):
            m = jnp.max(g, axis=0, keepdims=True)
            c = jnp.min(jnp.where(g == m, iota_c, BIG), axis=0, keepdims=True)
            cht_ref[j, :] = c[0]
            g = jnp.where(iota_c == c, jnp.float32(-1.0), g)


def _final_body(cand_ref, ch_ref, tv_ref, ti_ref):
    v = cand_ref[...]
    ch = ch_ref[...]
    lanes = lax.broadcasted_iota(jnp.int32, (BATCH, K * CH), 1)
    slot = lanes // CH
    within = lanes - slot * CH
    base = jnp.zeros((BATCH, K * CH), jnp.int32)
    for j in range(K):
        base = jnp.where(slot == j, ch[:, j:j + 1], base)
    gidx = base * CH + within
    out_lanes = lax.broadcasted_iota(jnp.int32, (BATCH, K), 1)
    tv = jnp.zeros((BATCH, K), jnp.float32)
    ti = jnp.zeros((BATCH, K), jnp.int32)
    for j in range(K):
        m = jnp.max(v, axis=1, keepdims=True)
        gi = jnp.min(jnp.where(v == m, gidx, BIG), axis=1, keepdims=True)
        tv = jnp.where(out_lanes == j, m, tv)
        ti = jnp.where(out_lanes == j, gi, ti)
        v = jnp.where(gidx == gi, jnp.float32(-1.0), v)
    tv_ref[...] = tv
    ti_ref[...] = ti


def _sc_gather(table, idx):
    info = plsc.get_sparse_core_info()
    nw = info.num_cores * info.num_subcores
    b = idx.shape[0]
    bpw = b // nw
    mesh = plsc.VectorSubcoreMesh(core_axis_name="c", subcore_axis_name="s")

    @functools.partial(
        pl.kernel, mesh=mesh,
        out_type=jax.ShapeDtypeStruct((b, CH), jnp.float32),
        scratch_types=[
            pltpu.VMEM((bpw,), jnp.int32),
            pltpu.VMEM((bpw, CH), jnp.float32),
            pltpu.SemaphoreType.DMA,
        ],
    )
    def k(table_hbm, idx_hbm, out_hbm, idx_v, rows_v, sem):
        wid = lax.axis_index("s") * info.num_cores + lax.axis_index("c")
        base = wid * bpw
        pltpu.sync_copy(idx_hbm.at[pl.ds(base, bpw)], idx_v)
        pltpu.async_copy(table_hbm.at[idx_v], rows_v, sem).wait()
        pltpu.sync_copy(rows_v, out_hbm.at[pl.ds(base, bpw)])

    return k(table, idx)


def kernel(entity_vectors, query_entities, k):
    evp = jnp.pad(entity_vectors, ((0, NPAD - N_ENT), (0, 0)))
    qv = jnp.take(evp, query_entities, axis=0)

    sim, cht = pl.pallas_call(
        _sim_body,
        grid=(GRID,),
        in_specs=[
            pl.BlockSpec((BATCH, N_REL), lambda i: (0, 0)),
            pl.BlockSpec((NB, N_REL), lambda i: (i, 0)),
        ],
        out_specs=[
            pl.BlockSpec((GPB, BATCH, CH), lambda i: (i, 0, 0)),
            pl.BlockSpec((CROWS, BATCH), lambda i: (0, 0)),
        ],
        out_shape=[
            jax.ShapeDtypeStruct((C, BATCH, CH), jnp.float32),
            jax.ShapeDtypeStruct((CROWS, BATCH), jnp.int32),
        ],
        scratch_shapes=[pltpu.VMEM((C, BATCH), jnp.float32)],
    )(qv, evp)

    ch = cht.T
    rowids = (ch[:, :K] * BATCH
              + jnp.arange(BATCH, dtype=jnp.int32)[:, None]).reshape(-1)
    cand = _sc_gather(sim.reshape(C * BATCH, CH), rowids)

    tv, ti = pl.pallas_call(
        _final_body,
        out_shape=[
            jax.ShapeDtypeStruct((BATCH, K), jnp.float32),
            jax.ShapeDtypeStruct((BATCH, K), jnp.int32),
        ],
    )(cand.reshape(BATCH, K * CH), ch)

    return tv, ti


# speedup vs baseline: 1.1125x; 1.1125x over previous
import functools

import jax
import jax.numpy as jnp
from jax import lax
from jax.experimental import pallas as pl
from jax.experimental.pallas import tpu as pltpu
from jax.experimental.pallas import tpu_sc as plsc

N_ENT = 100000
N_REL = 16
BATCH = 1024
K = 15
CH = 128
NPAD = 102400
C = NPAD // CH
NB = 4096
GRID = NPAD // NB
GPB = NB // CH
BIG = 1 << 30
CROWS = 16


def _sim_body(q_ref, e_ref, sim_ref, cht_ref, gms_ref):
    i = pl.program_id(0)
    q = q_ref[...]
    e = e_ref[...]
    s = lax.dot_general(q, e, (((1,), (1,)), ((), ())),
                        preferred_element_type=jnp.float32)
    for c in range(GPB):
        sim_ref[c] = s[:, c * CH:(c + 1) * CH]
    st = lax.dot_general(e, q, (((1,), (1,)), ((), ())),
                         preferred_element_type=jnp.float32)
    gms_ref[pl.ds(i * GPB, GPB), :] = st.reshape(GPB, CH, BATCH).max(axis=1)

    @pl.when(i == GRID - 1)
    def _chunksel():
        g = gms_ref[...]
        iota_c = lax.broadcasted_iota(jnp.int32, (C, BATCH), 0)
        for j in range(CROWS):
            m = jnp.max(g, axis=0, keepdims=True)
            c = jnp.min(jnp.where(g == m, iota_c, BIG), axis=0, keepdims=True)
            cht_ref[j, :] = c[0]
            g = jnp.where(iota_c == c, jnp.float32(-1.0), g)


def _final_body(cand_ref, ch_ref, tv_ref, ti_ref):
    v = cand_ref[...]
    ch = ch_ref[...]
    lanes = lax.broadcasted_iota(jnp.int32, (BATCH, K * CH), 1)
    slot = lanes // CH
    within = lanes - slot * CH
    base = jnp.zeros((BATCH, K * CH), jnp.int32)
    for j in range(K):
        base = jnp.where(slot == j, ch[:, j:j + 1], base)
    gidx = base * CH + within
    out_lanes = lax.broadcasted_iota(jnp.int32, (BATCH, K), 1)
    tv = jnp.zeros((BATCH, K), jnp.float32)
    ti = jnp.zeros((BATCH, K), jnp.int32)
    for j in range(K):
        m = jnp.max(v, axis=1, keepdims=True)
        gi = jnp.min(jnp.where(v == m, gidx, BIG), axis=1, keepdims=True)
        tv = jnp.where(out_lanes == j, m, tv)
        ti = jnp.where(out_lanes == j, gi, ti)
        v = jnp.where(gidx == gi, jnp.float32(-1.0), v)
    tv_ref[...] = tv
    ti_ref[...] = ti


def _sc_gather(table, idx):
    info = plsc.get_sparse_core_info()
    nw = info.num_cores * info.num_subcores
    b = idx.shape[0]
    bpw = b // nw
    mesh = plsc.VectorSubcoreMesh(core_axis_name="c", subcore_axis_name="s")

    @functools.partial(
        pl.kernel, mesh=mesh,
        out_type=jax.ShapeDtypeStruct((b, CH), jnp.float32),
        scratch_types=[
            pltpu.VMEM((bpw,), jnp.int32),
            pltpu.VMEM((bpw, CH), jnp.float32),
            pltpu.SemaphoreType.DMA,
        ],
    )
    def k(table_hbm, idx_hbm, out_hbm, idx_v, rows_v, sem):
        wid = lax.axis_index("s") * info.num_cores + lax.axis_index("c")
        base = wid * bpw
        pltpu.sync_copy(idx_hbm.at[pl.ds(base, bpw)], idx_v)
        pltpu.async_copy(table_hbm.at[idx_v], rows_v, sem).wait()
        pltpu.sync_copy(rows_v, out_hbm.at[pl.ds(base, bpw)])

    return k(table, idx)


def kernel(entity_vectors, query_entities, k):
    evp = jnp.pad(entity_vectors, ((0, NPAD - N_ENT), (0, 0)))
    qv = jnp.take(evp, query_entities, axis=0)

    sim, cht = pl.pallas_call(
        _sim_body,
        grid=(GRID,),
        in_specs=[
            pl.BlockSpec((BATCH, N_REL), lambda i: (0, 0)),
            pl.BlockSpec((NB, N_REL), lambda i: (i, 0)),
        ],
        out_specs=[
            pl.BlockSpec((GPB, BATCH, CH), lambda i: (i, 0, 0)),
            pl.BlockSpec((CROWS, BATCH), lambda i: (0, 0)),
        ],
        out_shape=[
            jax.ShapeDtypeStruct((C, BATCH, CH), jnp.float32),
            jax.ShapeDtypeStruct((CROWS, BATCH), jnp.int32),
        ],
        scratch_shapes=[pltpu.VMEM((C, BATCH), jnp.float32)],
    )(qv, evp)

    ch = cht.T
    rowids = (ch[:, :K] * BATCH
              + jnp.arange(BATCH, dtype=jnp.int32)[:, None]).reshape(-1)
    cand = _sc_gather(sim.reshape(C * BATCH, CH), rowids)

    tv, ti = pl.pallas_call(
        _final_body,
        out_shape=[
            jax.ShapeDtypeStruct((BATCH, K), jnp.float32),
            jax.ShapeDtypeStruct((BATCH, K), jnp.int32),
        ],
    )(cand.reshape(BATCH, K * CH), ch)

    return tv, ti
